# Initial kernel scaffold; baseline (speedup 1.0000x reference)
import jax
import jax.numpy as jnp
from jax.experimental import pallas as pl


def kernel(embedding, seg_gt):
    raise NotImplementedError("write your pallas kernel here")


# speedup vs baseline: 36.1216x; 36.1216x over previous
import functools

import jax
import jax.numpy as jnp
from jax import lax
from jax.experimental import pallas as pl
from jax.experimental.pallas import tpu as pltpu
from jax.experimental.pallas import tpu_sc as plsc

BS = 8
C = 8
N = 262144
L = 5
NC = 2
NS = 16
NW = NC * NS
PW = N // NW
CHUNK = 2048
NCHUNK = PW // CHUNK
NVEC = CHUNK // 16
DELTA_V = 0.5
DELTA_D = 3.0
ROWW = 48

_mesh = plsc.VectorSubcoreMesh(core_axis_name="core", subcore_axis_name="sub")
_PAIRS = [(1, 2), (1, 3), (1, 4), (2, 3), (2, 4), (3, 4)]


def _wid():
    return lax.axis_index("sub") * NC + lax.axis_index("core")


_params = pltpu.CompilerParams(needs_layout_passes=False)


def _vsum(v):
    return jnp.sum(v)


def _srecip(x):
    return (1.0 / jnp.full((16,), x, jnp.float32))[0]


def _vrsqrt(x):
    i = plsc.bitcast(x, jnp.int32)
    i = jnp.int32(0x5F3759DF) - lax.shift_right_logical(i, 1)
    y = plsc.bitcast(i, jnp.float32)
    xh = 0.5 * x
    for _ in range(3):
        y = y * (1.5 - xh * y * y)
    return y


@functools.partial(
    pl.kernel,
    out_type=jax.ShapeDtypeStruct((NW, BS, ROWW), jnp.float32),
    mesh=_mesh,
    compiler_params=_params,
    scratch_types=[
        pltpu.VMEM((C, CHUNK), jnp.float32),
        pltpu.VMEM((CHUNK,), jnp.int32),
        pltpu.VMEM((BS, ROWW), jnp.float32),
    ],
)
def _moments(emb_hbm, seg_hbm, out_hbm, emb_v, seg_v, row_v):
    base = _wid() * PW
    zvec = jnp.zeros((16,), jnp.float32)
    iota = lax.iota(jnp.int32, 16)

    def batch_body(b, carry):
        def chunk_body(ch, accs):
            start = base + ch * CHUNK
            pltpu.sync_copy(emb_hbm.at[b, :, pl.ds(start, CHUNK)], emb_v)
            pltpu.sync_copy(seg_hbm.at[b, pl.ds(start, CHUNK)], seg_v)

            def vec_body(i, accs):
                accs = list(accs)
                off = i * 16
                seg = seg_v[pl.ds(off, 16)]
                fs = [jnp.where(seg == l, 1.0, 0.0) for l in range(1, L)]
                for c in range(C):
                    x = emb_v[c, pl.ds(off, 16)]
                    for li in range(4):
                        accs[li * C + c] = accs[li * C + c] + fs[li] * x
                for li in range(4):
                    accs[32 + li] = accs[32 + li] + fs[li]
                return tuple(accs)

            return lax.fori_loop(0, NVEC, vec_body, accs)

        accs = lax.fori_loop(0, NCHUNK, chunk_body, (zvec,) * 36)
        rows = [zvec, zvec, zvec]
        for li in range(4):
            for c in range(C):
                j = (li + 1) * C + c
                s = _vsum(accs[li * C + c])
                rows[j // 16] = jnp.where(iota == j % 16, s, rows[j // 16])
        for li in range(4):
            j = 41 + li
            s = _vsum(accs[32 + li])
            rows[j // 16] = jnp.where(iota == j % 16, s, rows[j // 16])
        for k in range(3):
            row_v[b, pl.ds(k * 16, 16)] = rows[k]
        return carry

    lax.fori_loop(0, BS, batch_body, 0)
    pltpu.sync_copy(row_v, out_hbm.at[_wid()])


@functools.partial(
    pl.kernel,
    out_type=jax.ShapeDtypeStruct((NW, 16), jnp.float32),
    mesh=_mesh,
    compiler_params=_params,
    scratch_types=[
        pltpu.VMEM((C, CHUNK), jnp.float32),
        pltpu.VMEM((CHUNK,), jnp.int32),
        pltpu.VMEM((NW, BS, ROWW), jnp.float32),
        pltpu.VMEM((BS, ROWW), jnp.float32),
        pltpu.VMEM((BS, ROWW), jnp.float32),
        pltpu.VMEM((BS, 16), jnp.float32),
        pltpu.VMEM((16,), jnp.float32),
    ],
)
def _hinge(emb_hbm, seg_hbm, part_hbm, out_hbm, emb_v, seg_v, part_v,
           sums_v, mu_v, coef_v, row_v):
    wid = _wid()
    base = wid * PW
    zvec = jnp.zeros((16,), jnp.float32)
    iota = lax.iota(jnp.int32, 16)
    lo8 = iota < 8

    pltpu.sync_copy(part_hbm, part_v)
    for b in range(BS):
        def red_body(w, vs):
            return (vs[0] + part_v[w, b, pl.ds(0, 16)],
                    vs[1] + part_v[w, b, pl.ds(16, 16)],
                    vs[2] + part_v[w, b, pl.ds(32, 16)])
        v0, v1, v2 = lax.fori_loop(0, NW, red_body, (zvec, zvec, zvec))
        sums_v[b, pl.ds(0, 16)] = v0
        sums_v[b, pl.ds(16, 16)] = v1
        sums_v[b, pl.ds(32, 16)] = v2

    dsq_rows = [zvec, zvec, zvec]
    w_rows = [zvec, zvec, zvec]
    for b in range(BS):
        v0 = sums_v[b, pl.ds(0, 16)]
        v1 = sums_v[b, pl.ds(16, 16)]
        v2 = sums_v[b, pl.ds(32, 16)]
        cnts = [v2[8 + l] for l in range(1, L)]
        safes = [jnp.maximum(cn, 1.0) for cn in cnts]
        invs = [_srecip(s) for s in safes]
        pres = [jnp.where(cn > 0.0, 1.0, 0.0) for cn in cnts]
        nl = pres[0] + pres[1] + pres[2] + pres[3]
        m0 = v0 * invs[0]
        m1 = v1 * jnp.where(lo8, invs[1], invs[2])
        m2 = v2 * jnp.where(lo8, invs[3], 0.0)
        mu_v[b, pl.ds(0, 16)] = m0
        mu_v[b, pl.ds(16, 16)] = m1
        mu_v[b, pl.ds(32, 16)] = m2
        mrows = [m0, m1, m2]
        inv_nl = _srecip(jnp.maximum(nl, 1.0))
        crow = zvec
        for li in range(4):
            cf = pres[li] * invs[li] * inv_nl * (1.0 / BS)
            crow = jnp.where(iota == li + 1, cf, crow)
        coef_v[b, pl.ds(0, 16)] = crow
        guard = jnp.where(nl > 1.0, 1.0, 0.0)
        invd = guard * _srecip(jnp.maximum(nl * (nl - 1.0), 1.0)) * (1.0 / BS)
        for p, (i, j) in enumerate(_PAIRS):
            dsq = jnp.float32(0.0)
            for c in range(C):
                ji, jj = i * C + c, j * C + c
                d = mrows[ji // 16][ji % 16] - mrows[jj // 16][jj % 16]
                dsq = dsq + d * d
            e = b * 6 + p
            dsq_rows[e // 16] = jnp.where(iota == e % 16, dsq, dsq_rows[e // 16])
            wv = pres[i - 1] * pres[j - 1] * invd
            w_rows[e // 16] = jnp.where(iota == e % 16, wv, w_rows[e // 16])

    dist = jnp.float32(0.0)
    for k in range(3):
        dsq = jnp.maximum(dsq_rows[k], 1e-24)
        pd = dsq * _vrsqrt(dsq)
        h = jnp.maximum(DELTA_D - pd, 0.0)
        dist = dist + _vsum(w_rows[k] * h * h)

    def batch_body(b, var_w):
        bvec = jnp.full((16,), 0, jnp.int32) + b

        def chunk_body(ch, accs):
            start = base + ch * CHUNK
            pltpu.sync_copy(emb_hbm.at[b, :, pl.ds(start, CHUNK)], emb_v)
            pltpu.sync_copy(seg_hbm.at[b, pl.ds(start, CHUNK)], seg_v)

            def vec_body(i, accs):
                off = i * 16
                seg = seg_v[pl.ds(off, 16)]
                idx8 = seg * 8
                sq = jnp.zeros((16,), jnp.float32)
                for c in range(C):
                    x = emb_v[c, pl.ds(off, 16)]
                    mu = plsc.load_gather(mu_v, [bvec, idx8 + c])
                    d = x - mu
                    sq = sq + d * d
                norm = sq * _vrsqrt(sq)
                t = jnp.maximum(norm - DELTA_V, 0.0)
                h = t * t
                accs = list(accs)
                for li in range(4):
                    f = jnp.where(seg == li + 1, 1.0, 0.0)
                    accs[li] = accs[li] + f * h
                return tuple(accs)

            return lax.fori_loop(0, NVEC, vec_body, accs)

        accs = lax.fori_loop(0, NCHUNK, chunk_body, (zvec,) * 4)
        cvec = coef_v[b, pl.ds(0, 16)]
        for li in range(4):
            var_w = var_w + _vsum(accs[li]) * cvec[li + 1]
        return var_w

    var_w = lax.fori_loop(0, BS, batch_body, jnp.float32(0.0))

    row = (jnp.where(iota == 0, var_w, 0.0)
           + jnp.where(iota == 1, dist, 0.0)).astype(jnp.float32)
    row_v[...] = row
    pltpu.sync_copy(row_v, out_hbm.at[wid])


def kernel(embedding, seg_gt):
    partials = _moments(embedding, seg_gt)
    out = _hinge(embedding, seg_gt, partials)
    var_loss = jnp.sum(out[:, 0])
    dist_loss = out[0, 1]
    return (var_loss, dist_loss, jnp.zeros((), jnp.float32))
